# initial kernel scaffold (unmeasured)
import jax
import jax.numpy as jnp
from jax import lax
from jax.experimental import pallas as pl
from jax.experimental.pallas import tpu as pltpu


def kernel(
    x,
):
    def body(*refs):
        pass

    out_shape = jax.ShapeDtypeStruct(..., jnp.float32)
    return pl.pallas_call(body, out_shape=out_shape)(...)


# baseline (device time: 4277067 ns/iter reference)
import jax
import jax.numpy as jnp
from jax import lax
from jax.experimental import pallas as pl
from jax.experimental.pallas import tpu as pltpu

N_DEV = 16
LG_NDEV = 4
T = 2048
LG_T = 11
NCOL = 512
N_PRESORT_PASSES = LG_T * (LG_T + 1) // 2
N_MERGE_PASSES = sum(range(LG_T + 1, LG_T + LG_NDEV + 1))


def _cmpex_tile(xv, iota, lgj, asc):
    j = jnp.int32(1) << lgj
    mask_low = ((iota >> lgj) & 1) == 0
    up = pltpu.roll(xv, T - j, 0)
    down = pltpu.roll(xv, j, 0)
    partner = jnp.where(mask_low, up, down)
    mn = jnp.minimum(xv, partner)
    mx = jnp.maximum(xv, partner)
    keep_min = mask_low == asc
    return jnp.where(keep_min, mn, mx)


def _presort(x_ref, gbuf, my, iota):
    flip = (my & 1).astype(jnp.int32)
    xv0 = x_ref[...].astype(jnp.bfloat16)

    def step(_, carry):
        lgk, lgj, xv = carry
        asc = (((iota >> lgk) & 1) ^ flip) == 0
        xv = _cmpex_tile(xv, iota, lgj, asc)
        done = lgj == 0
        return (
            jnp.where(done, lgk + 1, lgk),
            jnp.where(done, lgk, lgj - 1),
            xv,
        )

    _, _, xv = lax.fori_loop(
        0, N_PRESORT_PASSES, step, (jnp.int32(1), jnp.int32(0), xv0)
    )
    gbuf[my] = xv


def _merge(gbuf, iota):

    def step(_, carry):
        lgk, lgj = carry

        @pl.when(lgj >= LG_T)
        def _():
            b = lgj - LG_T
            one = jnp.int32(1)

            def pair(q, c):
                t = ((q >> b) << (b + 1)) | (q & ((one << b) - 1))
                p = t | (one << b)
                lo = gbuf[t]
                hi = gbuf[p]
                mn = jnp.minimum(lo, hi)
                mx = jnp.maximum(lo, hi)
                asc = (((t << LG_T) >> lgk) & 1) == 0
                gbuf[t] = jnp.where(asc, mn, mx)
                gbuf[p] = jnp.where(asc, mx, mn)
                return c

            lax.fori_loop(0, N_DEV // 2, pair, 0)

        @pl.when(lgj < LG_T)
        def _():
            def tile(t, c):
                asc = (((t << LG_T) >> lgk) & 1) == 0
                gbuf[t] = _cmpex_tile(gbuf[t], iota, lgj, asc)
                return c

            lax.fori_loop(0, N_DEV, tile, 0)

        done = lgj == 0
        return jnp.where(done, lgk + 1, lgk), jnp.where(done, lgk, lgj - 1)

    lax.fori_loop(
        0, N_MERGE_PASSES, step, (jnp.int32(LG_T + 1), jnp.int32(LG_T))
    )


def _body(x_ref, o_ref, gbuf, send_sems, recv_sems):
    my = lax.axis_index("i")
    left = lax.rem(my + N_DEV - 1, N_DEV)
    right = lax.rem(my + 1, N_DEV)

    barrier_sem = pltpu.get_barrier_semaphore()
    for nbr in (left, right):
        pl.semaphore_signal(
            barrier_sem, inc=1,
            device_id=(nbr,), device_id_type=pl.DeviceIdType.MESH,
        )
    pl.semaphore_wait(barrier_sem, 2)

    iota = lax.broadcasted_iota(jnp.int32, (T, 1), 0)
    _presort(x_ref, gbuf, my, iota)

    for h in range(N_DEV - 1):
        slot = lax.rem(my - h + N_DEV, N_DEV)
        rdma = pltpu.make_async_remote_copy(
            src_ref=gbuf.at[slot],
            dst_ref=gbuf.at[slot],
            send_sem=send_sems.at[h],
            recv_sem=recv_sems.at[h],
            device_id=(right,),
            device_id_type=pl.DeviceIdType.MESH,
        )
        rdma.start()
        rdma.wait()

    _merge(gbuf, iota)
    o_ref[...] = gbuf[my].astype(jnp.float32)


def kernel(x):
    return pl.pallas_call(
        _body,
        out_shape=jax.ShapeDtypeStruct((T, NCOL), jnp.float32),
        in_specs=[pl.BlockSpec(memory_space=pltpu.VMEM)],
        out_specs=pl.BlockSpec(memory_space=pltpu.VMEM),
        scratch_shapes=[
            pltpu.VMEM((N_DEV, T, NCOL), jnp.bfloat16),
            pltpu.SemaphoreType.DMA((N_DEV - 1,)),
            pltpu.SemaphoreType.DMA((N_DEV - 1,)),
        ],
        compiler_params=pltpu.CompilerParams(
            collective_id=0, vmem_limit_bytes=100 * 1024 * 1024
        ),
    )(x)


# device time: 833323 ns/iter; 5.1325x vs baseline; 5.1325x over previous
import jax
import jax.numpy as jnp
from jax import lax
from jax.experimental import pallas as pl
from jax.experimental.pallas import tpu as pltpu

N_DEV = 16
LG_NDEV = 4
T = 2048
LG_T = 11
NCOL = 512
N_PRESORT_PASSES = LG_T * (LG_T + 1) // 2
N_EXCHANGES = LG_NDEV * (LG_NDEV + 1) // 2


def _cmpex_tile(xv, iota, lgj, asc):
    j = jnp.int32(1) << lgj
    mask_low = ((iota >> lgj) & 1) == 0
    up = pltpu.roll(xv, T - j, 0)
    down = pltpu.roll(xv, j, 0)
    partner = jnp.where(mask_low, up, down)
    mn = jnp.minimum(xv, partner)
    mx = jnp.maximum(xv, partner)
    keep_min = mask_low == asc
    return jnp.where(keep_min, mn, mx)


def _presort(x_ref, my, iota):
    flip = (my & 1).astype(jnp.int32)
    xv0 = x_ref[...].astype(jnp.bfloat16)

    def step(_, carry):
        lgk, lgj, xv = carry
        asc = (((iota >> lgk) & 1) ^ flip) == 0
        xv = _cmpex_tile(xv, iota, lgj, asc)
        done = lgj == 0
        return (
            jnp.where(done, lgk + 1, lgk),
            jnp.where(done, lgk, lgj - 1),
            xv,
        )

    _, _, xv = lax.fori_loop(
        0, N_PRESORT_PASSES, step, (jnp.int32(1), jnp.int32(0), xv0)
    )
    return xv


def _body(x_ref, o_ref, wbuf, pbuf, send_sems, recv_sems):
    my = lax.axis_index("i")
    iota = lax.broadcasted_iota(jnp.int32, (T, 1), 0)

    barrier_sem = pltpu.get_barrier_semaphore()
    for b in range(LG_NDEV):
        pl.semaphore_signal(
            barrier_sem, inc=1,
            device_id=(my ^ (1 << b),), device_id_type=pl.DeviceIdType.MESH,
        )
    pl.semaphore_wait(barrier_sem, LG_NDEV)

    xv = _presort(x_ref, my, iota)

    e = 0
    for lgk in range(LG_T + 1, LG_T + LG_NDEV + 1):
        asc = ((my >> (lgk - LG_T)) & 1) == 0

        for b in range(lgk - LG_T - 1, -1, -1):
            wbuf[...] = xv
            partner = my ^ (1 << b)
            rdma = pltpu.make_async_remote_copy(
                src_ref=wbuf,
                dst_ref=pbuf.at[e],
                send_sem=send_sems.at[e],
                recv_sem=recv_sems.at[e],
                device_id=(partner,),
                device_id_type=pl.DeviceIdType.MESH,
            )
            rdma.start()
            rdma.wait()
            pv = pbuf[e]
            low = ((my >> b) & 1) == 0
            keep_min = low == asc
            xv = jnp.where(
                keep_min, jnp.minimum(xv, pv), jnp.maximum(xv, pv)
            )
            e += 1

        def lstep(_, carry, asc=asc):
            lgj, yv = carry
            return lgj - 1, _cmpex_tile(yv, iota, lgj, asc)

        _, xv = lax.fori_loop(0, LG_T, lstep, (jnp.int32(LG_T - 1), xv))

    o_ref[...] = xv.astype(jnp.float32)


def kernel(x):
    return pl.pallas_call(
        _body,
        out_shape=jax.ShapeDtypeStruct((T, NCOL), jnp.float32),
        in_specs=[pl.BlockSpec(memory_space=pltpu.VMEM)],
        out_specs=pl.BlockSpec(memory_space=pltpu.VMEM),
        scratch_shapes=[
            pltpu.VMEM((T, NCOL), jnp.bfloat16),
            pltpu.VMEM((N_EXCHANGES, T, NCOL), jnp.bfloat16),
            pltpu.SemaphoreType.DMA((N_EXCHANGES,)),
            pltpu.SemaphoreType.DMA((N_EXCHANGES,)),
        ],
        compiler_params=pltpu.CompilerParams(
            collective_id=0, vmem_limit_bytes=100 * 1024 * 1024
        ),
    )(x)


# device time: 769938 ns/iter; 5.5551x vs baseline; 1.0823x over previous
import jax
import jax.numpy as jnp
from jax import lax
from jax.experimental import pallas as pl
from jax.experimental.pallas import tpu as pltpu

N_DEV = 16
LG_NDEV = 4
T = 2048
LG_T = 11
NCOL = 512
HALF = NCOL // 2
N_PRESORT_PASSES = LG_T * (LG_T + 1) // 2
N_EXCHANGES = LG_NDEV * (LG_NDEV + 1) // 2

EXCHANGES = [
    (lgk, b)
    for lgk in range(LG_T + 1, LG_T + LG_NDEV + 1)
    for b in range(lgk - LG_T - 1, -1, -1)
]


def _cmpex_tile(xv, iota, lgj, asc):
    j = jnp.int32(1) << lgj
    mask_low = ((iota >> lgj) & 1) == 0
    up = pltpu.roll(xv, T - j, 0)
    down = pltpu.roll(xv, j, 0)
    partner = jnp.where(mask_low, up, down)
    mn = jnp.minimum(xv, partner)
    mx = jnp.maximum(xv, partner)
    keep_min = mask_low == asc
    return jnp.where(keep_min, mn, mx)


def _presort(xv0, my, iota):
    flip = (my & 1).astype(jnp.int32)

    def step(_, carry):
        lgk, lgj, xv = carry
        asc = (((iota >> lgk) & 1) ^ flip) == 0
        xv = _cmpex_tile(xv, iota, lgj, asc)
        done = lgj == 0
        return (
            jnp.where(done, lgk + 1, lgk),
            jnp.where(done, lgk, lgj - 1),
            xv,
        )

    _, _, xv = lax.fori_loop(
        0, N_PRESORT_PASSES, step, (jnp.int32(1), jnp.int32(0), xv0)
    )
    return xv


def _body(
    x_ref, o_ref,
    wbuf_a, wbuf_b, pbuf_a, pbuf_b,
    ssem_a, rsem_a, ssem_b, rsem_b,
):
    my = lax.axis_index("i")
    iota = lax.broadcasted_iota(jnp.int32, (T, 1), 0)

    barrier_sem = pltpu.get_barrier_semaphore()
    for b in range(LG_NDEV):
        pl.semaphore_signal(
            barrier_sem, inc=1,
            device_id=(my ^ (1 << b),), device_id_type=pl.DeviceIdType.MESH,
        )
    pl.semaphore_wait(barrier_sem, LG_NDEV)

    def start(e, wbuf, pbuf, ssem, rsem, xv):
        _, b = EXCHANGES[e]
        wbuf[...] = xv
        rdma = pltpu.make_async_remote_copy(
            src_ref=wbuf,
            dst_ref=pbuf.at[e],
            send_sem=ssem.at[e],
            recv_sem=rsem.at[e],
            device_id=(my ^ (1 << b),),
            device_id_type=pl.DeviceIdType.MESH,
        )
        rdma.start()
        return rdma

    def finish(e, rdma, pbuf, xv):
        lgk, b = EXCHANGES[e]
        rdma.wait()
        pv = pbuf[e]
        low = ((my >> b) & 1) == 0
        asc = ((my >> (lgk - LG_T)) & 1) == 0
        keep_min = low == asc
        return jnp.where(keep_min, jnp.minimum(xv, pv), jnp.maximum(xv, pv))

    def locals_(lgk, xv):
        asc = ((my >> (lgk - LG_T)) & 1) == 0

        def lstep(_, carry):
            lgj, yv = carry
            return lgj - 1, _cmpex_tile(yv, iota, lgj, asc)

        _, xv = lax.fori_loop(0, LG_T, lstep, (jnp.int32(LG_T - 1), xv))
        return xv

    xa = _presort(x_ref[:, :HALF].astype(jnp.bfloat16), my, iota)
    ra = start(0, wbuf_a, pbuf_a, ssem_a, rsem_a, xa)
    xb = _presort(x_ref[:, HALF:].astype(jnp.bfloat16), my, iota)
    rb = start(0, wbuf_b, pbuf_b, ssem_b, rsem_b, xb)

    for e in range(N_EXCHANGES):
        lgk, b = EXCHANGES[e]
        xa = finish(e, ra, pbuf_a, xa)
        if b == 0:
            xa = locals_(lgk, xa)
        if e + 1 < N_EXCHANGES:
            ra = start(e + 1, wbuf_a, pbuf_a, ssem_a, rsem_a, xa)
        xb = finish(e, rb, pbuf_b, xb)
        if b == 0:
            xb = locals_(lgk, xb)
        if e + 1 < N_EXCHANGES:
            rb = start(e + 1, wbuf_b, pbuf_b, ssem_b, rsem_b, xb)

    o_ref[:, :HALF] = xa.astype(jnp.float32)
    o_ref[:, HALF:] = xb.astype(jnp.float32)


def kernel(x):
    return pl.pallas_call(
        _body,
        out_shape=jax.ShapeDtypeStruct((T, NCOL), jnp.float32),
        in_specs=[pl.BlockSpec(memory_space=pltpu.VMEM)],
        out_specs=pl.BlockSpec(memory_space=pltpu.VMEM),
        scratch_shapes=[
            pltpu.VMEM((T, HALF), jnp.bfloat16),
            pltpu.VMEM((T, HALF), jnp.bfloat16),
            pltpu.VMEM((N_EXCHANGES, T, HALF), jnp.bfloat16),
            pltpu.VMEM((N_EXCHANGES, T, HALF), jnp.bfloat16),
            pltpu.SemaphoreType.DMA((N_EXCHANGES,)),
            pltpu.SemaphoreType.DMA((N_EXCHANGES,)),
            pltpu.SemaphoreType.DMA((N_EXCHANGES,)),
            pltpu.SemaphoreType.DMA((N_EXCHANGES,)),
        ],
        compiler_params=pltpu.CompilerParams(
            collective_id=0, vmem_limit_bytes=100 * 1024 * 1024
        ),
    )(x)


# device time: 487787 ns/iter; 8.7683x vs baseline; 1.5784x over previous
import jax
import jax.numpy as jnp
from jax import lax
from jax.experimental import pallas as pl
from jax.experimental.pallas import tpu as pltpu

N_DEV = 16
LG_NDEV = 4
T = 2048
LG_T = 11
NCOL = 512
HALF = NCOL // 2
PW = 128
N_EXCHANGES = LG_NDEV * (LG_NDEV + 1) // 2
FIRST_LGK = LG_T + 1


def _asc_pass(v, lgj):
    j = 1 << lgj
    g = T // (2 * j)
    z = v.reshape(g, 2, j, PW)
    a = z[:, 0]
    b = z[:, 1]
    mn = jnp.minimum(a, b)[:, None]
    mx = jnp.maximum(a, b)[:, None]
    return jnp.concatenate([mn, mx], axis=1).reshape(T, PW)


def _rev_rows(v, lgn):
    rows = v.shape[0]
    for lv in range(lgn):
        z = v.reshape(rows >> (lv + 1), 2, 1 << lv, *v.shape[1:])
        v = jnp.concatenate([z[:, 1:2], z[:, 0:1]], axis=1).reshape(v.shape)
    return v


def _rev_second_run(v, lgrun):
    r = 1 << lgrun
    g = T // (2 * r)
    z = v.reshape(g, 2, r, PW)
    a = z[:, :1]
    b = _rev_rows(z[:, 1].reshape(g * r, PW), lgrun).reshape(g, 1, r, PW)
    return jnp.concatenate([a, b], axis=1).reshape(T, PW)


def _presort_panel(v):
    for lgrun in range(LG_T):
        v = _rev_second_run(v, lgrun)
        for lgj in range(lgrun, -1, -1):
            v = _asc_pass(v, lgj)
    return v


def _dir_pass(v, lgj, asc):
    j = 1 << lgj
    g = T // (2 * j)
    z = v.reshape(g, 2, j, PW)
    a = z[:, 0]
    b = z[:, 1]
    mn = jnp.minimum(a, b)
    mx = jnp.maximum(a, b)
    lo = jnp.where(asc, mn, mx)[:, None]
    hi = jnp.where(asc, mx, mn)[:, None]
    return jnp.concatenate([lo, hi], axis=1).reshape(T, PW)


def _local_merge(v, asc):
    for lgj in range(LG_T - 1, -1, -1):
        v = _dir_pass(v, lgj, asc)
    return v


def _body(
    x_ref, o_ref,
    wbuf_a, wbuf_b, pbuf_a, pbuf_b,
    ssem_a, rsem_a, ssem_b, rsem_b,
):
    my = lax.axis_index("i")

    barrier_sem = pltpu.get_barrier_semaphore()
    for b in range(LG_NDEV):
        pl.semaphore_signal(
            barrier_sem, inc=1,
            device_id=(my ^ (1 << b),), device_id_type=pl.DeviceIdType.MESH,
        )
    pl.semaphore_wait(barrier_sem, LG_NDEV)

    odd = (my & 1) == 1

    def presort_panel(pi, c):
        v = _presort_panel(x_ref[:, pl.ds(pi * PW, PW)].astype(jnp.bfloat16))
        rv = _rev_rows(v, LG_T)
        v = jnp.where(odd, rv, v)

        @pl.when(pi < HALF // PW)
        def _():
            wbuf_a[:, pl.ds(pi * PW, PW)] = v

        @pl.when(pi >= HALF // PW)
        def _():
            wbuf_b[:, pl.ds((pi - HALF // PW) * PW, PW)] = v

        return c

    lax.fori_loop(0, NCOL // PW, presort_panel, 0)

    def make_rdma(e, b, wbuf, pbuf, ssem, rsem):
        return pltpu.make_async_remote_copy(
            src_ref=wbuf,
            dst_ref=pbuf.at[e],
            send_sem=ssem.at[e],
            recv_sem=rsem.at[e],
            device_id=(my ^ (jnp.int32(1) << b),),
            device_id_type=pl.DeviceIdType.MESH,
        )

    def combine(e, lgk, b, wbuf, pbuf):
        low = ((my >> b) & 1) == 0
        asc = ((my >> (lgk - LG_T)) & 1) == 0
        keep_min = low == asc

        def panel(pi, c):
            sl = (slice(None), pl.ds(pi * PW, PW))
            xv = wbuf[sl]
            pv = pbuf[(e, *sl)]
            wbuf[sl] = jnp.where(
                keep_min, jnp.minimum(xv, pv), jnp.maximum(xv, pv)
            )
            return c

        lax.fori_loop(0, HALF // PW, panel, 0)

    def locals_(lgk, wbuf):
        asc = ((my >> (lgk - LG_T)) & 1) == 0

        def panel(pi, c):
            sl = (slice(None), pl.ds(pi * PW, PW))
            wbuf[sl] = _local_merge(wbuf[sl], asc)
            return c

        lax.fori_loop(0, HALF // PW, panel, 0)

    make_rdma(0, jnp.int32(0), wbuf_a, pbuf_a, ssem_a, rsem_a).start()
    make_rdma(0, jnp.int32(0), wbuf_b, pbuf_b, ssem_b, rsem_b).start()

    def exchange_step(e, carry):
        lgk, b = carry
        done = b == 0
        nlgk = jnp.where(done, lgk + 1, lgk)
        nb = jnp.where(done, lgk + 1 - FIRST_LGK, b - 1)

        def one(wbuf, pbuf, ssem, rsem):
            make_rdma(e, b, wbuf, pbuf, ssem, rsem).wait()
            combine(e, lgk, b, wbuf, pbuf)

            @pl.when(done)
            def _():
                locals_(lgk, wbuf)

            @pl.when(e + 1 < N_EXCHANGES)
            def _():
                make_rdma(e + 1, nb, wbuf, pbuf, ssem, rsem).start()

        one(wbuf_a, pbuf_a, ssem_a, rsem_a)
        one(wbuf_b, pbuf_b, ssem_b, rsem_b)
        return nlgk, nb

    lax.fori_loop(
        0, N_EXCHANGES, exchange_step,
        (jnp.int32(FIRST_LGK), jnp.int32(0)),
    )

    o_ref[:, :HALF] = wbuf_a[...].astype(jnp.float32)
    o_ref[:, HALF:] = wbuf_b[...].astype(jnp.float32)


def kernel(x):
    return pl.pallas_call(
        _body,
        out_shape=jax.ShapeDtypeStruct((T, NCOL), jnp.float32),
        in_specs=[pl.BlockSpec(memory_space=pltpu.VMEM)],
        out_specs=pl.BlockSpec(memory_space=pltpu.VMEM),
        scratch_shapes=[
            pltpu.VMEM((T, HALF), jnp.bfloat16),
            pltpu.VMEM((T, HALF), jnp.bfloat16),
            pltpu.VMEM((N_EXCHANGES, T, HALF), jnp.bfloat16),
            pltpu.VMEM((N_EXCHANGES, T, HALF), jnp.bfloat16),
            pltpu.SemaphoreType.DMA((N_EXCHANGES,)),
            pltpu.SemaphoreType.DMA((N_EXCHANGES,)),
            pltpu.SemaphoreType.DMA((N_EXCHANGES,)),
            pltpu.SemaphoreType.DMA((N_EXCHANGES,)),
        ],
        compiler_params=pltpu.CompilerParams(
            collective_id=0, vmem_limit_bytes=100 * 1024 * 1024
        ),
    )(x)


# device time: 454140 ns/iter; 9.4179x vs baseline; 1.0741x over previous
import jax
import jax.numpy as jnp
from jax import lax
from jax.experimental import pallas as pl
from jax.experimental.pallas import tpu as pltpu

N_DEV = 16
LG_NDEV = 4
T = 2048
LG_T = 11
NCOL = 512
NINST = 4
PW = NCOL // NINST
N_EXCHANGES = LG_NDEV * (LG_NDEV + 1) // 2
FIRST_LGK = LG_T + 1


def _asc_pass(v, lgj):
    j = 1 << lgj
    g = T // (2 * j)
    z = v.reshape(g, 2, j, PW)
    a = z[:, 0]
    b = z[:, 1]
    mn = jnp.minimum(a, b)[:, None]
    mx = jnp.maximum(a, b)[:, None]
    return jnp.concatenate([mn, mx], axis=1).reshape(T, PW)


def _rev_rows(v, lgn):
    rows = v.shape[0]
    for lv in range(lgn):
        z = v.reshape(rows >> (lv + 1), 2, 1 << lv, *v.shape[1:])
        v = jnp.concatenate([z[:, 1:2], z[:, 0:1]], axis=1).reshape(v.shape)
    return v


def _rev_second_run(v, lgrun):
    r = 1 << lgrun
    g = T // (2 * r)
    z = v.reshape(g, 2, r, PW)
    a = z[:, :1]
    b = _rev_rows(z[:, 1].reshape(g * r, PW), lgrun).reshape(g, 1, r, PW)
    return jnp.concatenate([a, b], axis=1).reshape(T, PW)


def _presort_panel(v):
    for lgrun in range(LG_T):
        v = _rev_second_run(v, lgrun)
        for lgj in range(lgrun, -1, -1):
            v = _asc_pass(v, lgj)
    return v


def _dir_pass(v, lgj, asc):
    j = 1 << lgj
    g = T // (2 * j)
    z = v.reshape(g, 2, j, PW)
    a = z[:, 0]
    b = z[:, 1]
    mn = jnp.minimum(a, b)
    mx = jnp.maximum(a, b)
    lo = jnp.where(asc, mn, mx)[:, None]
    hi = jnp.where(asc, mx, mn)[:, None]
    return jnp.concatenate([lo, hi], axis=1).reshape(T, PW)


def _local_merge(v, asc):
    for lgj in range(LG_T - 1, -1, -1):
        v = _dir_pass(v, lgj, asc)
    return v


def _body(x_ref, o_ref, wbuf, pbuf, ssem, rsem):
    my = lax.axis_index("i")

    barrier_sem = pltpu.get_barrier_semaphore()
    for b in range(LG_NDEV):
        pl.semaphore_signal(
            barrier_sem, inc=1,
            device_id=(my ^ (1 << b),), device_id_type=pl.DeviceIdType.MESH,
        )
    pl.semaphore_wait(barrier_sem, LG_NDEV)

    odd = (my & 1) == 1

    def presort_panel(pi, c):
        v = _presort_panel(x_ref[:, pl.ds(pi * PW, PW)].astype(jnp.bfloat16))
        rv = _rev_rows(v, LG_T)
        wbuf[pi] = jnp.where(odd, rv, v)
        return c

    lax.fori_loop(0, NINST, presort_panel, 0)

    def make_rdma(i, e, b):
        return pltpu.make_async_remote_copy(
            src_ref=wbuf.at[i],
            dst_ref=pbuf.at[i, e],
            send_sem=ssem.at[i, e],
            recv_sem=rsem.at[i, e],
            device_id=(my ^ (jnp.int32(1) << b),),
            device_id_type=pl.DeviceIdType.MESH,
        )

    for i in range(NINST):
        make_rdma(i, 0, jnp.int32(0)).start()

    def exchange_step(e, carry):
        lgk, b = carry
        done = b == 0
        nlgk = jnp.where(done, lgk + 1, lgk)
        nb = jnp.where(done, lgk + 1 - FIRST_LGK, b - 1)

        low = ((my >> b) & 1) == 0
        asc = ((my >> (lgk - LG_T)) & 1) == 0
        keep_min = low == asc

        for i in range(NINST):
            make_rdma(i, e, b).wait()
            xv = wbuf[i]
            pv = pbuf[i, e]
            xv = jnp.where(
                keep_min, jnp.minimum(xv, pv), jnp.maximum(xv, pv)
            )
            wbuf[i] = xv

            @pl.when(done)
            def _(i=i, asc=asc):
                wbuf[i] = _local_merge(wbuf[i], asc)

            @pl.when(e + 1 < N_EXCHANGES)
            def _(i=i, nb=nb):
                make_rdma(i, e + 1, nb).start()

        return nlgk, nb

    lax.fori_loop(
        0, N_EXCHANGES, exchange_step,
        (jnp.int32(FIRST_LGK), jnp.int32(0)),
    )

    for i in range(NINST):
        o_ref[:, i * PW:(i + 1) * PW] = wbuf[i].astype(jnp.float32)


def kernel(x):
    return pl.pallas_call(
        _body,
        out_shape=jax.ShapeDtypeStruct((T, NCOL), jnp.float32),
        in_specs=[pl.BlockSpec(memory_space=pltpu.VMEM)],
        out_specs=pl.BlockSpec(memory_space=pltpu.VMEM),
        scratch_shapes=[
            pltpu.VMEM((NINST, T, PW), jnp.bfloat16),
            pltpu.VMEM((NINST, N_EXCHANGES, T, PW), jnp.bfloat16),
            pltpu.SemaphoreType.DMA((NINST, N_EXCHANGES)),
            pltpu.SemaphoreType.DMA((NINST, N_EXCHANGES)),
        ],
        compiler_params=pltpu.CompilerParams(
            collective_id=0, vmem_limit_bytes=100 * 1024 * 1024
        ),
    )(x)


# device time: 421933 ns/iter; 10.1368x vs baseline; 1.0763x over previous
import jax
import jax.numpy as jnp
from jax import lax
from jax.experimental import pallas as pl
from jax.experimental.pallas import tpu as pltpu

N_DEV = 16
LG_NDEV = 4
T = 2048
LG_T = 11
NCOL = 512
NINST = 4
PW = NCOL // NINST
N_EXCHANGES = LG_NDEV * (LG_NDEV + 1) // 2
FIRST_LGK = LG_T + 1


def _asc_pass(v, lgj):
    j = 1 << lgj
    g = T // (2 * j)
    z = v.reshape(g, 2, j, PW)
    a = z[:, 0]
    b = z[:, 1]
    mn = jnp.minimum(a, b)[:, None]
    mx = jnp.maximum(a, b)[:, None]
    return jnp.concatenate([mn, mx], axis=1).reshape(T, PW)


def _rev_rows(v, lgn):
    rows = v.shape[0]
    for lv in range(lgn):
        z = v.reshape(rows >> (lv + 1), 2, 1 << lv, *v.shape[1:])
        v = jnp.concatenate([z[:, 1:2], z[:, 0:1]], axis=1).reshape(v.shape)
    return v


def _rev_second_run(v, lgrun):
    r = 1 << lgrun
    g = T // (2 * r)
    z = v.reshape(g, 2, r, PW)
    a = z[:, :1]
    b = _rev_rows(z[:, 1].reshape(g * r, PW), lgrun).reshape(g, 1, r, PW)
    return jnp.concatenate([a, b], axis=1).reshape(T, PW)


def _presort_panel(v):
    for lgrun in range(LG_T):
        v = _rev_second_run(v, lgrun)
        for lgj in range(lgrun, -1, -1):
            v = _asc_pass(v, lgj)
    return v


def _dir_pass(v, lgj, asc):
    j = 1 << lgj
    g = T // (2 * j)
    z = v.reshape(g, 2, j, PW)
    a = z[:, 0]
    b = z[:, 1]
    mn = jnp.minimum(a, b)
    mx = jnp.maximum(a, b)
    lo = jnp.where(asc, mn, mx)[:, None]
    hi = jnp.where(asc, mx, mn)[:, None]
    return jnp.concatenate([lo, hi], axis=1).reshape(T, PW)


def _local_merge(v, asc):
    for lgj in range(LG_T - 1, -1, -1):
        v = _dir_pass(v, lgj, asc)
    return v


def _body(x_ref, o_ref, wbuf, pbuf, ssem, rsem):
    my = lax.axis_index("i")

    barrier_sem = pltpu.get_barrier_semaphore()
    for b in range(LG_NDEV):
        pl.semaphore_signal(
            barrier_sem, inc=1,
            device_id=(my ^ (1 << b),), device_id_type=pl.DeviceIdType.MESH,
        )
    pl.semaphore_wait(barrier_sem, LG_NDEV)

    odd = (my & 1) == 1

    def presort_panel(pi, c):
        v = _presort_panel(x_ref[:, pl.ds(pi * PW, PW)].astype(jnp.bfloat16))
        rv = _rev_rows(v, LG_T)
        wbuf[pi] = jnp.where(odd, rv, v)
        return c

    lax.fori_loop(0, NINST, presort_panel, 0)

    def make_rdma(i, e, b):
        return pltpu.make_async_remote_copy(
            src_ref=wbuf.at[i],
            dst_ref=pbuf.at[i, e],
            send_sem=ssem.at[i, e],
            recv_sem=rsem.at[i, e],
            device_id=(my ^ (jnp.int32(1) << b),),
            device_id_type=pl.DeviceIdType.MESH,
        )

    for i in range(NINST):
        make_rdma(i, 0, jnp.int32(0)).start()

    def lgk_b_of(e):
        k2 = (
            (e >= 1).astype(jnp.int32)
            + (e >= 3).astype(jnp.int32)
            + (e >= 6).astype(jnp.int32)
        )
        return FIRST_LGK + k2, k2 * (k2 + 1) // 2 + k2 - e

    def exchange_step(s, carry):
        for i in range(NINST):
            e = s - i

            @pl.when((e >= 0) & (e < N_EXCHANGES))
            def _(i=i, e=e):
                lgk, b = lgk_b_of(e)
                low = ((my >> b) & 1) == 0
                asc = ((my >> (lgk - LG_T)) & 1) == 0
                keep_min = low == asc
                make_rdma(i, e, b).wait()
                xv = wbuf[i]
                pv = pbuf[i, e]
                wbuf[i] = jnp.where(
                    keep_min, jnp.minimum(xv, pv), jnp.maximum(xv, pv)
                )

                @pl.when(b == 0)
                def _():
                    wbuf[i] = _local_merge(wbuf[i], asc)

                @pl.when(e + 1 < N_EXCHANGES)
                def _():
                    _, nb = lgk_b_of(e + 1)
                    make_rdma(i, e + 1, nb).start()

        return carry

    lax.fori_loop(0, N_EXCHANGES + NINST - 1, exchange_step, 0)

    for i in range(NINST):
        o_ref[:, i * PW:(i + 1) * PW] = wbuf[i].astype(jnp.float32)


def kernel(x):
    return pl.pallas_call(
        _body,
        out_shape=jax.ShapeDtypeStruct((T, NCOL), jnp.float32),
        in_specs=[pl.BlockSpec(memory_space=pltpu.VMEM)],
        out_specs=pl.BlockSpec(memory_space=pltpu.VMEM),
        scratch_shapes=[
            pltpu.VMEM((NINST, T, PW), jnp.bfloat16),
            pltpu.VMEM((NINST, N_EXCHANGES, T, PW), jnp.bfloat16),
            pltpu.SemaphoreType.DMA((NINST, N_EXCHANGES)),
            pltpu.SemaphoreType.DMA((NINST, N_EXCHANGES)),
        ],
        compiler_params=pltpu.CompilerParams(
            collective_id=0, vmem_limit_bytes=100 * 1024 * 1024
        ),
    )(x)
